# Initial kernel scaffold; baseline (speedup 1.0000x reference)
import jax
import jax.numpy as jnp
from jax.experimental import pallas as pl


def kernel(x):
    raise NotImplementedError("write your pallas kernel here")


# speedup vs baseline: 10.9535x; 10.9535x over previous
import jax
import jax.numpy as jnp
from jax.experimental import pallas as pl
from jax.experimental.pallas import tpu as pltpu

_N = 65536
_D = 64
_K = 512
_ITERS = 4
_T = 2048
_NT = _N // _T

_F32 = jnp.float32


def _dot(a, b, dims):
    return jax.lax.dot_general(a, b, (dims, ((), ())),
                               preferred_element_type=_F32)


def _body(x_ref, out_ref, centers_ref, sums_ref, counts_ref, acc_ref):
    centers_ref[...] = x_ref[0:_K, :]
    acc_ref[...] = jnp.zeros((1, 1), _F32)
    ones_t = jnp.ones((_T, 1), _F32)
    ones_d = jnp.ones((1, _D), _F32)
    lane_iota = jax.lax.broadcasted_iota(jnp.int32, (_T, _K), 1)

    for p in range(_ITERS + 1):
        final = p == _ITERS
        c = centers_ref[...]
        cn = _dot(ones_d, c * c, ((1,), (1,)))
        sums_ref[...] = jnp.zeros((_K, _D), _F32)
        counts_ref[...] = jnp.zeros((_K, 1), _F32)

        def tile(t, carry):
            xt = x_ref[pl.ds(t * _T, _T), :]
            logits = _dot(xt, c, ((1,), (1,)))
            xn = jnp.sum(xt * xt, axis=1, keepdims=True)
            d2 = (xn - 2.0 * logits) + cn
            cid = jnp.argmin(d2, axis=1).astype(jnp.int32).reshape(_T, 1)
            oh = (cid == lane_iota).astype(_F32)
            sums_ref[...] += _dot(oh, xt, ((0,), (0,)))
            counts_ref[...] += _dot(oh, ones_t, ((0,), (0,)))
            if final:
                m = jnp.max(logits, axis=1, keepdims=True)
                lse = m + jnp.log(
                    jnp.sum(jnp.exp(logits - m), axis=1, keepdims=True))
                acc_ref[...] = acc_ref[...] + jnp.sum(lse)
            return carry

        jax.lax.fori_loop(0, _NT, tile, 0)

        if not final:
            cnt = counts_ref[...]
            newc = sums_ref[...] / jnp.maximum(cnt, 1.0)
            centers_ref[...] = jnp.where(cnt > 0.0, newc, c)

    cnt = counts_ref[...]
    occ = (cnt > 0.0).astype(_F32)
    ki = jax.lax.broadcasted_iota(jnp.int32, (_K, _K), 0)
    ji = jax.lax.broadcasted_iota(jnp.int32, (_K, _K), 1)
    tril = (ji < ki).astype(_F32)
    rank = _dot(tril, occ, ((1,), (0,)))
    rank_i = rank.astype(jnp.int32)
    oh_rank = (rank_i == ji).astype(_F32)
    c_rank = _dot(oh_rank, centers_ref[...], ((1,), (0,)))
    picked_sum = jnp.sum(sums_ref[...] * c_rank)
    out_ref[...] = (acc_ref[...] - picked_sum) / _N


def kernel(x):
    out = pl.pallas_call(
        _body,
        out_shape=jax.ShapeDtypeStruct((1, 1), _F32),
        scratch_shapes=[
            pltpu.VMEM((_K, _D), _F32),
            pltpu.VMEM((_K, _D), _F32),
            pltpu.VMEM((_K, 1), _F32),
            pltpu.VMEM((1, 1), _F32),
        ],
    )(x)
    return out[0, 0]


# speedup vs baseline: 18.4184x; 1.6815x over previous
import jax
import jax.numpy as jnp
from jax.experimental import pallas as pl
from jax.experimental.pallas import tpu as pltpu

_N = 65536
_D = 64
_K = 512
_ITERS = 4
_T = 2048
_NT = _N // _T

_F32 = jnp.float32


def _dot(a, b, dims):
    return jax.lax.dot_general(a, b, (dims, ((), ())),
                               preferred_element_type=_F32)


def _body(x_ref, out_ref, centers_ref, sums_ref, counts_ref, acc_ref):
    centers_ref[...] = x_ref[0:_K, :]
    acc_ref[...] = jnp.zeros((1, 1), _F32)
    ones_d = jnp.ones((1, _D), _F32)

    for p in range(_ITERS + 1):
        final = p == _ITERS
        c = centers_ref[...]
        cn = _dot(ones_d, c * c, ((1,), (1,)))
        sums_ref[...] = jnp.zeros((_K, _D), _F32)
        counts_ref[...] = jnp.zeros((1, _K), _F32)

        def tile(t, carry):
            xt = x_ref[pl.ds(t * _T, _T), :]
            logits = _dot(xt, c, ((1,), (1,)))
            d2 = cn - 2.0 * logits
            rowmin = jnp.min(d2, axis=1, keepdims=True)
            oh = (d2 == rowmin).astype(_F32)
            sums_ref[...] += _dot(oh, xt, ((0,), (0,)))
            counts_ref[...] += jnp.sum(oh, axis=0, keepdims=True)
            if final:
                m = jnp.max(logits, axis=1, keepdims=True)
                lse = m + jnp.log(
                    jnp.sum(jnp.exp(logits - m), axis=1, keepdims=True))
                acc_ref[...] = acc_ref[...] + jnp.sum(lse)
            return carry

        jax.lax.fori_loop(0, _NT, tile, 0)

        if not final:
            cnt = counts_ref[...].reshape(_K, 1)
            newc = sums_ref[...] / jnp.maximum(cnt, 1.0)
            centers_ref[...] = jnp.where(cnt > 0.0, newc, c)

    cnt = counts_ref[...].reshape(_K, 1)
    occ = (cnt > 0.0).astype(_F32)
    ki = jax.lax.broadcasted_iota(jnp.int32, (_K, _K), 0)
    ji = jax.lax.broadcasted_iota(jnp.int32, (_K, _K), 1)
    tril = (ji < ki).astype(_F32)
    rank = _dot(tril, occ, ((1,), (0,)))
    rank_i = rank.astype(jnp.int32)
    oh_rank = (rank_i == ji).astype(_F32)
    c_rank = _dot(oh_rank, centers_ref[...], ((1,), (0,)))
    picked_sum = jnp.sum(sums_ref[...] * c_rank)
    out_ref[...] = (acc_ref[...] - picked_sum) / _N


def kernel(x):
    out = pl.pallas_call(
        _body,
        out_shape=jax.ShapeDtypeStruct((1, 1), _F32),
        scratch_shapes=[
            pltpu.VMEM((_K, _D), _F32),
            pltpu.VMEM((_K, _D), _F32),
            pltpu.VMEM((1, _K), _F32),
            pltpu.VMEM((1, 1), _F32),
        ],
    )(x)
    return out[0, 0]


# speedup vs baseline: 20.2283x; 1.0983x over previous
import jax
import jax.numpy as jnp
from jax.experimental import pallas as pl
from jax.experimental.pallas import tpu as pltpu

_N = 65536
_D = 64
_K = 512
_ITERS = 4
_T = 4096
_NT = _N // _T

_F32 = jnp.float32


def _dot(a, b, dims):
    return jax.lax.dot_general(a, b, (dims, ((), ())),
                               preferred_element_type=_F32)


def _body(x_ref, out_ref, centers_ref, sums_ref, counts_ref, acc_ref):
    centers_ref[...] = x_ref[0:_K, :]
    acc_ref[...] = jnp.zeros((1, 1), _F32)
    ones_d = jnp.ones((1, _D), _F32)

    for p in range(_ITERS + 1):
        final = p == _ITERS
        c = centers_ref[...]
        cm2 = -2.0 * c
        cn = _dot(ones_d, c * c, ((1,), (1,)))
        sums_ref[...] = jnp.zeros((_K, _D), _F32)
        counts_ref[...] = jnp.zeros((1, _K), _F32)

        def tile(t, carry):
            xt = x_ref[pl.ds(t * _T, _T), :]
            q = _dot(xt, cm2, ((1,), (1,)))
            d2 = q + cn
            rowmin = jnp.min(d2, axis=1, keepdims=True)
            oh = (d2 == rowmin).astype(_F32)
            sums_ref[...] += _dot(oh, xt, ((0,), (0,)))
            counts_ref[...] += jnp.sum(oh, axis=0, keepdims=True)
            if final:
                logits = -0.5 * q
                m = jnp.max(logits, axis=1, keepdims=True)
                lse = m + jnp.log(
                    jnp.sum(jnp.exp(logits - m), axis=1, keepdims=True))
                acc_ref[...] = acc_ref[...] + jnp.sum(lse)
            return carry

        jax.lax.fori_loop(0, _NT, tile, 0)

        if not final:
            cnt = counts_ref[...].reshape(_K, 1)
            newc = sums_ref[...] / jnp.maximum(cnt, 1.0)
            centers_ref[...] = jnp.where(cnt > 0.0, newc, c)

    cnt = counts_ref[...].reshape(_K, 1)
    occ = (cnt > 0.0).astype(_F32)
    ki = jax.lax.broadcasted_iota(jnp.int32, (_K, _K), 0)
    ji = jax.lax.broadcasted_iota(jnp.int32, (_K, _K), 1)
    tril = (ji < ki).astype(_F32)
    rank = _dot(tril, occ, ((1,), (0,)))
    rank_i = rank.astype(jnp.int32)
    oh_rank = (rank_i == ji).astype(_F32)
    c_rank = _dot(oh_rank, centers_ref[...], ((1,), (0,)))
    picked_sum = jnp.sum(sums_ref[...] * c_rank)
    out_ref[...] = (acc_ref[...] - picked_sum) / _N


def kernel(x):
    out = pl.pallas_call(
        _body,
        out_shape=jax.ShapeDtypeStruct((1, 1), _F32),
        scratch_shapes=[
            pltpu.VMEM((_K, _D), _F32),
            pltpu.VMEM((_K, _D), _F32),
            pltpu.VMEM((1, _K), _F32),
            pltpu.VMEM((1, 1), _F32),
        ],
    )(x)
    return out[0, 0]
